# baseline (speedup 1.0000x reference)
import functools
import math

import jax
import jax.numpy as jnp
from jax import lax
from jax.experimental import pallas as pl
from jax.experimental.pallas import tpu as pltpu
from jax.experimental.pallas import tpu_sc as plsc

D_MODEL = 1024
SCALE = math.sqrt(D_MODEL)

NUM_CORES = 2
NUM_SUBCORES = 16
NUM_WORKERS = NUM_CORES * NUM_SUBCORES
LANES = 16

CHUNK = 16
NBUF = 7
LOOKAHEAD = 4

B_TC = 4096
TC_GROUP = 256
TC_NBUF = 2


def _sc_part(x_flat, table, b_sc):
    b_per_w = b_sc // NUM_WORKERS
    n_chunks = b_per_w // CHUNK
    mesh = plsc.VectorSubcoreMesh(core_axis_name="c", subcore_axis_name="s")

    @functools.partial(
        pl.kernel,
        mesh=mesh,
        out_type=jax.ShapeDtypeStruct((b_sc, D_MODEL), jnp.float32),
        scratch_types=[
            pltpu.VMEM((b_per_w,), jnp.int32),
        ]
        + [pltpu.VMEM((CHUNK, D_MODEL), jnp.float32) for _ in range(NBUF)]
        + [pltpu.SemaphoreType.DMA for _ in range(2 * NBUF)],
    )
    def k(idx_hbm, table_hbm, out_hbm, idx_v, *bufs_and_sems):
        bufs = bufs_and_sems[:NBUF]
        sem_g = bufs_and_sems[NBUF : 2 * NBUF]
        sem_w = bufs_and_sems[2 * NBUF :]

        wid = lax.axis_index("s") * NUM_CORES + lax.axis_index("c")
        base = wid * b_per_w
        pltpu.sync_copy(idx_hbm.at[pl.ds(base, b_per_w)], idx_v)

        gather_h = [None] * n_chunks
        write_h = [None] * n_chunks
        HALF = CHUNK // 2

        def scale_half(buf, h):
            def row_body(r, carry):
                for j in range(D_MODEL // LANES):
                    sl = pl.ds(j * LANES, LANES)
                    buf[r, sl] = buf[r, sl] * SCALE
                return carry

            lax.fori_loop(h * HALF, (h + 1) * HALF, row_body, 0, unroll=False)

        for g in range(n_chunks + LOOKAHEAD):
            if g < n_chunks:
                s = g % NBUF
                if g >= NBUF:
                    for hcopy in write_h[g - NBUF]:
                        hcopy.wait()
                gather_h[g] = pltpu.async_copy(
                    table_hbm.at[idx_v.at[pl.ds(g * CHUNK, CHUNK)]],
                    bufs[s],
                    sem_g[s],
                )
            p = g - LOOKAHEAD
            if p >= 0:
                s = p % NBUF
                gather_h[p].wait()
                hs = []
                for h in range(2):
                    scale_half(bufs[s], h)
                    hs.append(
                        pltpu.async_copy(
                            bufs[s].at[pl.ds(h * HALF, HALF)],
                            out_hbm.at[pl.ds(base + p * CHUNK + h * HALF, HALF)],
                            sem_w[s],
                        )
                    )
                write_h[p] = hs
        for p in range(n_chunks - NBUF, n_chunks):
            for hcopy in write_h[p]:
                hcopy.wait()

    return k(x_flat, table)


def _tc_part(x_tc, table):
    n = x_tc.shape[0]
    n_groups = n // TC_GROUP

    def body(x_smem, table_hbm, out_hbm, buf, sem_in, sem_out):
        def issue_group(g, s):
            def row(j, carry):
                idx = x_smem[g * TC_GROUP + j]
                pltpu.make_async_copy(
                    table_hbm.at[pl.ds(idx, 1)],
                    buf.at[s].at[pl.ds(j, 1)],
                    sem_in.at[s],
                ).start()
                return carry

            lax.fori_loop(0, TC_GROUP, row, 0, unroll=4)

        def wait_group(s):
            pltpu.make_async_copy(
                table_hbm.at[pl.ds(0, TC_GROUP)], buf.at[s], sem_in.at[s]
            ).wait()

        def write_group(g, s):
            return pltpu.async_copy(
                buf.at[s], out_hbm.at[pl.ds(g * TC_GROUP, TC_GROUP)], sem_out.at[s]
            )

        write_h = [None] * n_groups
        for g in range(n_groups):
            s = g % TC_NBUF
            if g >= TC_NBUF:
                write_h[g - TC_NBUF].wait()
            issue_group(g, s)
            wait_group(s)
            buf[s] = buf[s] * SCALE
            write_h[g] = write_group(g, s)
        for g in range(n_groups - TC_NBUF, n_groups):
            if write_h[g] is not None:
                write_h[g].wait()

    return pl.pallas_call(
        body,
        in_specs=[
            pl.BlockSpec(memory_space=pltpu.MemorySpace.SMEM),
            pl.BlockSpec(memory_space=pltpu.MemorySpace.HBM),
        ],
        out_specs=pl.BlockSpec(memory_space=pltpu.MemorySpace.HBM),
        out_shape=jax.ShapeDtypeStruct((n, D_MODEL), jnp.float32),
        scratch_shapes=[
            pltpu.VMEM((TC_NBUF, TC_GROUP, D_MODEL), jnp.float32),
            pltpu.SemaphoreType.DMA((TC_NBUF,)),
            pltpu.SemaphoreType.DMA((TC_NBUF,)),
        ],
    )(x_tc, table)


@functools.partial(jax.jit, static_argnames=("b_total",))
def _scaled_embed(x_flat, table, b_total):
    b_sc = b_total - B_TC
    out_sc = _sc_part(x_flat[:b_sc], table, b_sc)
    out_tc = _tc_part(x_flat[b_sc:], table)
    return jnp.concatenate([out_sc, out_tc], axis=0)


def kernel(x, table):
    b_total = x.shape[0] * x.shape[1]
    x_flat = x.reshape(b_total).astype(jnp.int32)
    out = _scaled_embed(x_flat, table, b_total)
    return out.reshape(x.shape[0], x.shape[1], D_MODEL)
